# baseline (speedup 1.0000x reference)
import jax
import jax.numpy as jnp
from jax import lax
from jax.experimental import pallas as pl
from jax.experimental.pallas import tpu as pltpu
from jax.experimental.pallas import tpu_sc as plsc

NUM_POINTS = 1000000
DIMS = 64
BATCH = 16384

NC = 2
NS = 16
NW = NC * NS
BPW = BATCH // NW
QC = 128
NQ = BPW // QC
LANES = 16


def _sc_body(lidx_hbm, ridx_hbm, emb_hbm, blhs_hbm, brhs_hbm, out_hbm,
             lidx_v, ridx_v, lrows_v, rrows_v, lb_v, rb_v, m_v, out_v,
             sem, bsem):
    wid = lax.axis_index("s") * NC + lax.axis_index("c")
    base = wid * BPW

    pltpu.sync_copy(lidx_hbm.at[pl.ds(base, BPW)], lidx_v)
    pltpu.sync_copy(ridx_hbm.at[pl.ds(base, BPW)], ridx_v)

    bcopies = []
    for c in range(NQ):
        bcopies.append(pltpu.async_copy(
            blhs_hbm.at[lidx_v.at[pl.ds(c * QC, QC)]],
            lb_v.at[pl.ds(c * QC, QC)], bsem))
        bcopies.append(pltpu.async_copy(
            brhs_hbm.at[ridx_v.at[pl.ds(c * QC, QC)]],
            rb_v.at[pl.ds(c * QC, QC)], bsem))

    lane = lax.iota(jnp.int32, LANES)

    def quarter(q, carry):
        copies = []
        for blk in range(QC // LANES):
            ilv = lidx_v[pl.ds(q * QC + blk * LANES, LANES)]
            irv = ridx_v[pl.ds(q * QC + blk * LANES, LANES)]
            for j in range(LANES):
                p = blk * LANES + j
                copies.append(pltpu.async_copy(
                    emb_hbm.at[ilv[j]], lrows_v.at[p], sem))
                copies.append(pltpu.async_copy(
                    emb_hbm.at[irv[j]], rrows_v.at[p], sem))
        for cp in copies:
            cp.wait()
        for blk in range(QC // LANES):
            for j in range(LANES):
                p = blk * LANES + j
                acc = jnp.zeros((LANES,), jnp.float32)
                for k in range(DIMS // LANES):
                    lv = lrows_v[p, pl.ds(k * LANES, LANES)]
                    rv = rrows_v[p, pl.ds(k * LANES, LANES)]
                    d = lv - rv
                    acc = acc + d * d
                m_v[pl.ds(j * LANES, LANES)] = acc
            sqv = jnp.zeros((LANES,), jnp.float32)
            for k in range(LANES):
                sqv = sqv + plsc.load_gather(m_v, [lane * LANES + k])
            o = q * QC + blk * LANES
            out_v[pl.ds(o, LANES)] = (
                lb_v[pl.ds(o, LANES)] + rb_v[pl.ds(o, LANES)] - sqv)
        return carry

    for bc in bcopies:
        bc.wait()
    lax.fori_loop(0, NQ, quarter, 0)

    pltpu.sync_copy(out_v, out_hbm.at[pl.ds(base, BPW)])


TCOLS = 2048


def _transpose_body(x_ref, o_ref):
    o_ref[...] = x_ref[...].T


def _tc_transpose(emb_t):
    grid = (NUM_POINTS + TCOLS - 1) // TCOLS
    return pl.pallas_call(
        _transpose_body,
        grid=(grid,),
        in_specs=[pl.BlockSpec((DIMS, TCOLS), lambda i: (0, i))],
        out_specs=pl.BlockSpec((TCOLS, DIMS), lambda i: (i, 0)),
        out_shape=jax.ShapeDtypeStruct((NUM_POINTS, DIMS), jnp.float32),
        compiler_params=pltpu.CompilerParams(
            dimension_semantics=("arbitrary",)),
    )(emb_t)


@jax.jit
def _run(lidx, ridx, emb, bias_lhs, bias_rhs):
    mesh = plsc.VectorSubcoreMesh(core_axis_name="c", subcore_axis_name="s")
    f = pl.kernel(
        _sc_body,
        out_type=jax.ShapeDtypeStruct((BATCH,), jnp.float32),
        mesh=mesh,
        compiler_params=pltpu.CompilerParams(needs_layout_passes=False),
        scratch_types=[
            pltpu.VMEM((BPW,), jnp.int32),
            pltpu.VMEM((BPW,), jnp.int32),
            pltpu.VMEM((QC, DIMS), jnp.float32),
            pltpu.VMEM((QC, DIMS), jnp.float32),
            pltpu.VMEM((BPW,), jnp.float32),
            pltpu.VMEM((BPW,), jnp.float32),
            pltpu.VMEM((LANES * LANES,), jnp.float32),
            pltpu.VMEM((BPW,), jnp.float32),
            pltpu.SemaphoreType.DMA,
            pltpu.SemaphoreType.DMA,
        ],
    )
    return f(lidx, ridx, emb, bias_lhs, bias_rhs)


def kernel(input_triplet, embeddings, bias_lhs, bias_rhs):
    lidx = input_triplet[:, 0].astype(jnp.int32)
    ridx = input_triplet[:, -1].astype(jnp.int32)
    emb_rm = _tc_transpose(embeddings.T)
    return _run(lidx, ridx, emb_rm, bias_lhs, bias_rhs)


# speedup vs baseline: 1.3280x; 1.3280x over previous
import jax
import jax.numpy as jnp
from jax import lax
from jax.experimental import pallas as pl
from jax.experimental.pallas import tpu as pltpu
from jax.experimental.pallas import tpu_sc as plsc

NUM_POINTS = 1000000
DIMS = 64
BATCH = 16384

NC = 2
NS = 16
NW = NC * NS
BPW = BATCH // NW
QC = 128
NQ = BPW // QC
LANES = 16


def _sc_body(lidx_hbm, ridx_hbm, emb_hbm, blhs_hbm, brhs_hbm, out_hbm,
             lidx_v, ridx_v, lrows_v, rrows_v, lb_v, rb_v, m_v, out_v,
             sem, bsem):
    wid = lax.axis_index("s") * NC + lax.axis_index("c")
    base = wid * BPW

    pltpu.sync_copy(lidx_hbm.at[pl.ds(base, BPW)], lidx_v)
    pltpu.sync_copy(ridx_hbm.at[pl.ds(base, BPW)], ridx_v)

    bcopies = []
    for c in range(NQ):
        bcopies.append(pltpu.async_copy(
            blhs_hbm.at[lidx_v.at[pl.ds(c * QC, QC)]],
            lb_v.at[pl.ds(c * QC, QC)], bsem))
        bcopies.append(pltpu.async_copy(
            brhs_hbm.at[ridx_v.at[pl.ds(c * QC, QC)]],
            rb_v.at[pl.ds(c * QC, QC)], bsem))

    lane = lax.iota(jnp.int32, LANES)

    def quarter(q, carry):
        copies = []
        for blk in range(QC // LANES):
            ilv = lidx_v[pl.ds(q * QC + blk * LANES, LANES)]
            irv = ridx_v[pl.ds(q * QC + blk * LANES, LANES)]
            for j in range(LANES):
                p = blk * LANES + j
                copies.append(pltpu.async_copy(
                    emb_hbm.at[ilv[j]], lrows_v.at[p], sem))
                copies.append(pltpu.async_copy(
                    emb_hbm.at[irv[j]], rrows_v.at[p], sem))
        for cp in copies:
            cp.wait()
        for blk in range(QC // LANES):
            for j in range(LANES):
                p = blk * LANES + j
                acc = jnp.zeros((LANES,), jnp.float32)
                for k in range(DIMS // LANES):
                    lv = lrows_v[p, pl.ds(k * LANES, LANES)]
                    rv = rrows_v[p, pl.ds(k * LANES, LANES)]
                    d = lv - rv
                    acc = acc + d * d
                m_v[pl.ds(j * LANES, LANES)] = acc
            sqv = jnp.zeros((LANES,), jnp.float32)
            for k in range(LANES):
                sqv = sqv + plsc.load_gather(m_v, [lane * LANES + k])
            o = q * QC + blk * LANES
            out_v[pl.ds(o, LANES)] = (
                lb_v[pl.ds(o, LANES)] + rb_v[pl.ds(o, LANES)] - sqv)
        return carry

    for bc in bcopies:
        bc.wait()
    lax.fori_loop(0, NQ, quarter, 0)

    pltpu.sync_copy(out_v, out_hbm.at[pl.ds(base, BPW)])


@jax.jit
def _run(lidx, ridx, emb, bias_lhs, bias_rhs):
    mesh = plsc.VectorSubcoreMesh(core_axis_name="c", subcore_axis_name="s")
    f = pl.kernel(
        _sc_body,
        out_type=jax.ShapeDtypeStruct((BATCH,), jnp.float32),
        mesh=mesh,
        compiler_params=pltpu.CompilerParams(needs_layout_passes=False),
        scratch_types=[
            pltpu.VMEM((BPW,), jnp.int32),
            pltpu.VMEM((BPW,), jnp.int32),
            pltpu.VMEM((QC, DIMS), jnp.float32),
            pltpu.VMEM((QC, DIMS), jnp.float32),
            pltpu.VMEM((BPW,), jnp.float32),
            pltpu.VMEM((BPW,), jnp.float32),
            pltpu.VMEM((LANES * LANES,), jnp.float32),
            pltpu.VMEM((BPW,), jnp.float32),
            pltpu.SemaphoreType.DMA,
            pltpu.SemaphoreType.DMA,
        ],
    )
    return f(lidx, ridx, emb, bias_lhs, bias_rhs)


def kernel(input_triplet, embeddings, bias_lhs, bias_rhs):
    lidx = input_triplet[:, 0].astype(jnp.int32)
    ridx = input_triplet[:, -1].astype(jnp.int32)
    return _run(lidx, ridx, embeddings, bias_lhs, bias_rhs)
